# Initial kernel scaffold; baseline (speedup 1.0000x reference)
import jax
import jax.numpy as jnp
from jax.experimental import pallas as pl


def kernel(tensorInput, tensorVertical, tensorHorizontal, tensorOffsetX, tensorOffsetY, tensorMask):
    raise NotImplementedError("write your pallas kernel here")


# speedup vs baseline: 353.1501x; 353.1501x over previous
import functools

import jax
import jax.numpy as jnp
from jax import lax
from jax.experimental import pallas as pl
from jax.experimental.pallas import tpu as pltpu
from jax.experimental.pallas import tpu_sc as plsc

B, C, H, W = 2, 3, 256, 256
F = 5
F2 = F * F
HW = H * W
L = 16
NVEC = W // L

_MESH = dict(core_axis_name="c", subcore_axis_name="s", num_cores=2,
             num_subcores=16)


def _sc_body(inp, vert, horiz, offx, offy, mask, out,
             plane, dxb, dyb, mb, vb, hb, orow):
    b = lax.axis_index("c")
    s = lax.axis_index("s")

    @pl.when(s < 15)
    def _():
        ch = s // 5
        shard = s % 5
        pad = jnp.minimum(shard, 1)
        r0 = shard * 51 + pad
        nrows = 52 - pad

        pltpu.sync_copy(inp.at[b, ch], plane)

        xiota = lax.iota(jnp.int32, L).astype(jnp.float32)

        def row_body(r, carry):
            rw = r * W
            pltpu.sync_copy(offy.at[b, :, pl.ds(rw, W)], dxb)
            pltpu.sync_copy(offx.at[b, :, pl.ds(rw, W)], dyb)
            pltpu.sync_copy(mask.at[b, :, pl.ds(rw, W)], mb)
            pltpu.sync_copy(vert.at[b, :, pl.ds(rw, W)], vb)
            pltpu.sync_copy(horiz.at[b, :, pl.ds(rw, W)], hb)
            rf = r.astype(jnp.float32)

            def vec_body(v, carry2):
                base = v * L
                xoff = xiota + (base.astype(jnp.float32) - 1.5)

                def ti_body(ti, acc):
                    vtap = vb[ti, pl.ds(base, L)]
                    cy = rf + (ti.astype(jnp.float32) - 1.5)

                    def tj_body(tj, acc2):
                        t = ti * F + tj
                        dx = dxb[t, pl.ds(base, L)]
                        dy = dyb[t, pl.ds(base, L)]
                        m = mb[t, pl.ds(base, L)]
                        htap = hb[tj, pl.ds(base, L)]
                        ix = jnp.clip(dx + (xoff + tj.astype(jnp.float32)),
                                      0.0, W - 1.0)
                        iy = jnp.clip(dy + cy, 0.0, H - 1.0)
                        x0 = ix.astype(jnp.int32)
                        y0 = iy.astype(jnp.int32)
                        wx1 = ix - x0.astype(jnp.float32)
                        wy1 = iy - y0.astype(jnp.float32)
                        x1 = jnp.minimum(x0 + 1, W - 1)
                        y1 = jnp.minimum(y0 + 1, H - 1)
                        row0 = y0 * W
                        row1 = y1 * W
                        v00 = plsc.load_gather(plane, [row0 + x0])
                        v01 = plsc.load_gather(plane, [row0 + x1])
                        v10 = plsc.load_gather(plane, [row1 + x0])
                        v11 = plsc.load_gather(plane, [row1 + x1])
                        l0 = v00 + wx1 * (v01 - v00)
                        l1 = v10 + wx1 * (v11 - v10)
                        smp = l0 + wy1 * (l1 - l0)
                        wgt = vtap * htap * m
                        return acc2 + wgt * smp

                    return lax.fori_loop(0, F, tj_body, acc)

                acc = lax.fori_loop(0, F, ti_body,
                                    jnp.zeros((L,), jnp.float32))
                orow[pl.ds(base, L)] = acc
                return carry2

            lax.fori_loop(0, NVEC, vec_body, jnp.int32(0))
            pltpu.sync_copy(orow, out.at[b, ch, pl.ds(rw, W)])
            return carry

        lax.fori_loop(r0, r0 + nrows, row_body, jnp.int32(0))


def _build_sc_call():
    return pl.kernel(
        _sc_body,
        out_type=jax.ShapeDtypeStruct((B, C, HW), jnp.float32),
        mesh=plsc.VectorSubcoreMesh(**_MESH),
        scratch_types=[
            pltpu.VMEM((HW,), jnp.float32),
            pltpu.VMEM((F2, W), jnp.float32),
            pltpu.VMEM((F2, W), jnp.float32),
            pltpu.VMEM((F2, W), jnp.float32),
            pltpu.VMEM((F, W), jnp.float32),
            pltpu.VMEM((F, W), jnp.float32),
            pltpu.VMEM((W,), jnp.float32),
        ],
        compiler_params=pltpu.CompilerParams(use_tc_tiling_on_sc=False,
                                             needs_layout_passes=False),
    )


def kernel(tensorInput, tensorVertical, tensorHorizontal,
           tensorOffsetX, tensorOffsetY, tensorMask):
    inp = tensorInput.reshape(B, C, HW)
    vert = tensorVertical.reshape(B, F, HW)
    horiz = tensorHorizontal.reshape(B, F, HW)
    offx = tensorOffsetX.reshape(B, F2, HW)
    offy = tensorOffsetY.reshape(B, F2, HW)
    msk = tensorMask.reshape(B, F2, HW)
    out = _build_sc_call()(inp, vert, horiz, offx, offy, msk)
    return out.reshape(B, C, H, W)


# speedup vs baseline: 690.5371x; 1.9554x over previous
import jax
import jax.numpy as jnp
from jax import lax
from jax.experimental import pallas as pl
from jax.experimental.pallas import tpu as pltpu
from jax.experimental.pallas import tpu_sc as plsc

B, C, H, W = 2, 3, 256, 256
F = 5
F2 = F * F
HW = H * W
L = 16
NVEC = W // L

_MESH = dict(core_axis_name="c", subcore_axis_name="s", num_cores=2,
             num_subcores=16)


def _sc_body(inp, vert, horiz, offx, offy, mask, out,
             plane, dxb, dyb, mb, vb, hb, orow, sems):
    b = lax.axis_index("c")
    s = lax.axis_index("s")

    @pl.when(s < 15)
    def _():
        ch = s // 5
        shard = s % 5
        pad = jnp.minimum(shard, 1)
        r0 = shard * 51 + pad
        r1 = r0 + 52 - pad

        pltpu.sync_copy(inp.at[b, ch], plane)

        def issue(row, slot):
            rw = row * W
            sem = sems.at[slot]
            pltpu.async_copy(offy.at[b, :, pl.ds(rw, W)], dxb.at[slot], sem)
            pltpu.async_copy(offx.at[b, :, pl.ds(rw, W)], dyb.at[slot], sem)
            pltpu.async_copy(mask.at[b, :, pl.ds(rw, W)], mb.at[slot], sem)
            pltpu.async_copy(vert.at[b, :, pl.ds(rw, W)], vb.at[slot], sem)
            pltpu.async_copy(horiz.at[b, :, pl.ds(rw, W)], hb.at[slot], sem)

        def drain(slot):
            sem = sems.at[slot]
            z = pl.ds(0, W)
            pltpu.make_async_copy(offy.at[b, :, z], dxb.at[slot], sem).wait()
            pltpu.make_async_copy(offx.at[b, :, z], dyb.at[slot], sem).wait()
            pltpu.make_async_copy(mask.at[b, :, z], mb.at[slot], sem).wait()
            pltpu.make_async_copy(vert.at[b, :, z], vb.at[slot], sem).wait()
            pltpu.make_async_copy(horiz.at[b, :, z], hb.at[slot], sem).wait()

        xiota = lax.iota(jnp.int32, L).astype(jnp.float32)

        issue(r0, r0 & 1)

        def row_body(r, carry):
            slot = r & 1

            @pl.when(r + 1 < r1)
            def _():
                issue(r + 1, 1 - slot)

            drain(slot)
            rf = r.astype(jnp.float32)

            def vec_body(v, carry2):
                base = v * L
                xoff = xiota + (base.astype(jnp.float32) - 1.5)
                sl = pl.ds(base, L)
                xoffs = [xoff + float(tj) for tj in range(F)]
                vtaps = [vb[slot, ti, sl] for ti in range(F)]
                htaps = [hb[slot, tj, sl] for tj in range(F)]
                acc = jnp.zeros((L,), jnp.float32)
                for ti in range(F):
                    cy = rf + (float(ti) - 1.5)
                    for tj in range(F):
                        t = ti * F + tj
                        dx = dxb[slot, t, sl]
                        dy = dyb[slot, t, sl]
                        m = mb[slot, t, sl]
                        ix = jnp.clip(dx + xoffs[tj], 0.0, W - 1.0)
                        iy = jnp.clip(dy + cy, 0.0, H - 1.0)
                        x0 = ix.astype(jnp.int32)
                        y0 = iy.astype(jnp.int32)
                        wx1 = ix - x0.astype(jnp.float32)
                        wy1 = iy - y0.astype(jnp.float32)
                        x1 = jnp.minimum(x0 + 1, W - 1)
                        y1 = jnp.minimum(y0 + 1, H - 1)
                        row0 = y0 * W
                        row1 = y1 * W
                        v00 = plsc.load_gather(plane, [row0 + x0])
                        v01 = plsc.load_gather(plane, [row0 + x1])
                        v10 = plsc.load_gather(plane, [row1 + x0])
                        v11 = plsc.load_gather(plane, [row1 + x1])
                        l0 = v00 + wx1 * (v01 - v00)
                        l1 = v10 + wx1 * (v11 - v10)
                        smp = l0 + wy1 * (l1 - l0)
                        acc = acc + (vtaps[ti] * htaps[tj] * m) * smp
                orow[sl] = acc
                return carry2

            lax.fori_loop(0, NVEC, vec_body, jnp.int32(0))
            pltpu.sync_copy(orow, out.at[b, ch, pl.ds(r * W, W)])
            return carry

        lax.fori_loop(r0, r1, row_body, jnp.int32(0))


def _build_sc_call():
    return pl.kernel(
        _sc_body,
        out_type=jax.ShapeDtypeStruct((B, C, HW), jnp.float32),
        mesh=plsc.VectorSubcoreMesh(**_MESH),
        scratch_types=[
            pltpu.VMEM((HW,), jnp.float32),
            pltpu.VMEM((2, F2, W), jnp.float32),
            pltpu.VMEM((2, F2, W), jnp.float32),
            pltpu.VMEM((2, F2, W), jnp.float32),
            pltpu.VMEM((2, F, W), jnp.float32),
            pltpu.VMEM((2, F, W), jnp.float32),
            pltpu.VMEM((W,), jnp.float32),
            pltpu.SemaphoreType.DMA((2,)),
        ],
        compiler_params=pltpu.CompilerParams(use_tc_tiling_on_sc=False,
                                             needs_layout_passes=False),
    )


def kernel(tensorInput, tensorVertical, tensorHorizontal,
           tensorOffsetX, tensorOffsetY, tensorMask):
    inp = tensorInput.reshape(B, C, HW)
    vert = tensorVertical.reshape(B, F, HW)
    horiz = tensorHorizontal.reshape(B, F, HW)
    offx = tensorOffsetX.reshape(B, F2, HW)
    offy = tensorOffsetY.reshape(B, F2, HW)
    msk = tensorMask.reshape(B, F2, HW)
    out = _build_sc_call()(inp, vert, horiz, offx, offy, msk)
    return out.reshape(B, C, H, W)
